# baseline (speedup 1.0000x reference)
import functools

import jax
import jax.numpy as jnp
from jax.experimental import pallas as pl
from jax.experimental.pallas import tpu as pltpu

NUM_EXPERTS = 16
TOP_K = 2
AUX_COEFF = 0.01
CHUNK_T = 512
NBUF = 10


def _router_kernel(x_hbm, w_ref, ew_hbm, ei_hbm, aux_hbm,
                   buf, acc_ref, ew_v, ei_v, aux_v, sem, outsem,
                   *, n_tokens, n_chunks):
    k = pl.program_id(0)
    slot = jax.lax.rem(k, NBUF)

    @pl.when(k == 0)
    def _warmup():
        acc_ref[...] = jnp.zeros_like(acc_ref)
        for j in range(NBUF):
            pltpu.make_async_copy(x_hbm.at[j], buf.at[j], sem.at[j]).start()

    @pl.when(jnp.logical_and(k >= 1, k + NBUF - 1 < n_chunks))
    def _refill():
        idx = k + NBUF - 1
        s2 = jax.lax.rem(idx, NBUF)
        pltpu.make_async_copy(x_hbm.at[idx], buf.at[s2], sem.at[s2]).start()

    pltpu.make_async_copy(x_hbm.at[k], buf.at[slot], sem.at[slot]).wait()

    logits = jax.lax.dot_general(
        buf[slot], w_ref[...],
        dimension_numbers=(((1,), (1,)), ((), ())),
        preferred_element_type=jnp.float32,
    )

    lane = jax.lax.broadcasted_iota(jnp.int32, logits.shape, 1)

    m1 = jnp.max(logits, axis=1, keepdims=True)
    i1 = jnp.argmax(logits, axis=1).astype(jnp.int32)
    is_top1 = lane == i1[:, None]
    masked = jnp.where(is_top1, -jnp.inf, logits)
    m2 = jnp.max(masked, axis=1, keepdims=True)
    i2 = jnp.argmax(masked, axis=1).astype(jnp.int32)

    e2 = jnp.exp(m2 - m1)
    denom2 = 1.0 + e2
    row = k * CHUNK_T
    ew_v[pl.ds(row, CHUNK_T), :] = jnp.concatenate(
        [1.0 / denom2, e2 / denom2], axis=1)
    ei_v[pl.ds(row, CHUNK_T), :] = jnp.stack([i1, i2], axis=1)

    ex = jnp.exp(logits - m1)
    gates = ex / jnp.sum(ex, axis=1, keepdims=True)

    acc_ref[0:1, :] += jnp.sum(is_top1.astype(jnp.float32), axis=0, keepdims=True)
    acc_ref[1:2, :] += jnp.sum(gates, axis=0, keepdims=True)

    @pl.when(k == n_chunks - 1)
    def _finish():
        f = acc_ref[0:1, :] / n_tokens
        p = acc_ref[1:2, :] / n_tokens
        aux_v[...] = (AUX_COEFF * NUM_EXPERTS * jnp.sum(f * p)).reshape(1, 1)
        cp0 = pltpu.make_async_copy(ew_v, ew_hbm, outsem.at[0])
        cp1 = pltpu.make_async_copy(ei_v, ei_hbm, outsem.at[1])
        cp2 = pltpu.make_async_copy(aux_v, aux_hbm, outsem.at[2])
        cp0.start()
        cp1.start()
        cp2.start()
        cp0.wait()
        cp1.wait()
        cp2.wait()


def kernel(x, W):
    n_tokens, d_model = x.shape
    n_chunks = n_tokens // CHUNK_T
    x3 = x.reshape(n_chunks, CHUNK_T, d_model)

    ew, ei, aux = pl.pallas_call(
        functools.partial(_router_kernel, n_tokens=n_tokens, n_chunks=n_chunks),
        grid=(n_chunks,),
        in_specs=[
            pl.BlockSpec(memory_space=pltpu.MemorySpace.HBM),
            pl.BlockSpec((NUM_EXPERTS, d_model), lambda i: (0, 0)),
        ],
        out_specs=[
            pl.BlockSpec(memory_space=pltpu.MemorySpace.HBM),
            pl.BlockSpec(memory_space=pltpu.MemorySpace.HBM),
            pl.BlockSpec(memory_space=pltpu.MemorySpace.HBM),
        ],
        out_shape=[
            jax.ShapeDtypeStruct((n_tokens, TOP_K), jnp.float32),
            jax.ShapeDtypeStruct((n_tokens, TOP_K), jnp.int32),
            jax.ShapeDtypeStruct((1, 1), jnp.float32),
        ],
        scratch_shapes=[
            pltpu.VMEM((NBUF, CHUNK_T, d_model), jnp.float32),
            pltpu.VMEM((2, NUM_EXPERTS), jnp.float32),
            pltpu.VMEM((n_tokens, TOP_K), jnp.float32),
            pltpu.VMEM((n_tokens, TOP_K), jnp.int32),
            pltpu.VMEM((1, 1), jnp.float32),
            pltpu.SemaphoreType.DMA((NBUF,)),
            pltpu.SemaphoreType.DMA((3,)),
        ],
    )(x3, W)
    return ew, ei, aux.reshape(())


# speedup vs baseline: 1.3498x; 1.3498x over previous
import functools

import jax
import jax.numpy as jnp
from jax.experimental import pallas as pl
from jax.experimental.pallas import tpu as pltpu

NUM_EXPERTS = 16
TOP_K = 2
AUX_COEFF = 0.01
CHUNK_T = 512
NBUF = 12


def _router_kernel(x_hbm, w_ref, ew_ref, ei_ref, aux_ref, buf, acc_ref, sem,
                   *, n_tokens, n_chunks):
    k = pl.program_id(0)
    slot = jax.lax.rem(k, NBUF)

    @pl.when(k == 0)
    def _warmup():
        acc_ref[...] = jnp.zeros_like(acc_ref)
        for j in range(NBUF):
            pltpu.make_async_copy(x_hbm.at[j], buf.at[j], sem.at[j]).start()

    @pl.when(jnp.logical_and(k >= 1, k + NBUF - 1 < n_chunks))
    def _refill():
        idx = k + NBUF - 1
        s2 = jax.lax.rem(idx, NBUF)
        pltpu.make_async_copy(x_hbm.at[idx], buf.at[s2], sem.at[s2]).start()

    pltpu.make_async_copy(x_hbm.at[k], buf.at[slot], sem.at[slot]).wait()

    logits = jax.lax.dot_general(
        buf[slot], w_ref[...],
        dimension_numbers=(((1,), (1,)), ((), ())),
        preferred_element_type=jnp.float32,
    )

    lane = jax.lax.broadcasted_iota(jnp.int32, logits.shape, 1)

    m1 = jnp.max(logits, axis=1, keepdims=True)
    i1 = jnp.argmax(logits, axis=1).astype(jnp.int32)
    is_top1 = lane == i1[:, None]
    masked = jnp.where(is_top1, -jnp.inf, logits)
    m2 = jnp.max(masked, axis=1, keepdims=True)
    i2 = jnp.argmax(masked, axis=1).astype(jnp.int32)

    e2 = jnp.exp(m2 - m1)
    denom2 = 1.0 + e2
    ew_ref[...] = jnp.transpose(
        jnp.concatenate([1.0 / denom2, e2 / denom2], axis=1), (1, 0))
    ei_ref[...] = jnp.transpose(jnp.stack([i1, i2], axis=1), (1, 0))

    ex = jnp.exp(logits - m1)
    gates = ex / jnp.sum(ex, axis=1, keepdims=True)

    acc_ref[0:1, :] += jnp.sum(is_top1.astype(jnp.float32), axis=0, keepdims=True)
    acc_ref[1:2, :] += jnp.sum(gates, axis=0, keepdims=True)

    @pl.when(k == n_chunks - 1)
    def _finish():
        f = acc_ref[0:1, :] / n_tokens
        p = acc_ref[1:2, :] / n_tokens
        aux_ref[...] = (AUX_COEFF * NUM_EXPERTS * jnp.sum(f * p)).reshape(1, 1)


def kernel(x, W):
    n_tokens, d_model = x.shape
    n_chunks = n_tokens // CHUNK_T
    x3 = x.reshape(n_chunks, CHUNK_T, d_model)

    ew_t, ei_t, aux = pl.pallas_call(
        functools.partial(_router_kernel, n_tokens=n_tokens, n_chunks=n_chunks),
        grid=(n_chunks,),
        in_specs=[
            pl.BlockSpec(memory_space=pltpu.MemorySpace.HBM),
            pl.BlockSpec((NUM_EXPERTS, d_model), lambda i: (0, 0)),
        ],
        out_specs=[
            pl.BlockSpec((TOP_K, CHUNK_T), lambda i: (0, i)),
            pl.BlockSpec((TOP_K, CHUNK_T), lambda i: (0, i)),
            pl.BlockSpec((1, 1), lambda i: (0, 0)),
        ],
        out_shape=[
            jax.ShapeDtypeStruct((TOP_K, n_tokens), jnp.float32),
            jax.ShapeDtypeStruct((TOP_K, n_tokens), jnp.int32),
            jax.ShapeDtypeStruct((1, 1), jnp.float32),
        ],
        scratch_shapes=[
            pltpu.VMEM((NBUF, CHUNK_T, d_model), jnp.float32),
            pltpu.VMEM((2, NUM_EXPERTS), jnp.float32),
            pltpu.SemaphoreType.DMA((NBUF,)),
        ],
    )(x3, W)
    return ew_t.T, ei_t.T, aux.reshape(())
